# baseline (speedup 1.0000x reference)
import functools

import jax
import jax.numpy as jnp
import numpy as np
from jax import lax
from jax.experimental import pallas as pl
from jax.experimental.pallas import tpu as pltpu
from jax.experimental.pallas import tpu_sc as plsc

NUM_FIELDS = 26
VOCAB_PER_FIELD = 100000
EMBED_DIM = 16
BATCH = 16384
MLP_HIDDEN = 128
EMBED_OUT = NUM_FIELDS * EMBED_DIM
TOTAL_ROWS = NUM_FIELDS * VOCAB_PER_FIELD
BF = BATCH * NUM_FIELDS

NC = 2
NS = 16
NW = NC * NS

CCOL = 1024
NFULL = 2539
TAIL_OFF = NFULL * CCOL
TAILC = 64
ROWS_PAD = TOTAL_ROWS + TAILC
CHUNK_FLOATS = CCOL * EMBED_DIM


OUT_FLOATS = ROWS_PAD * EMBED_DIM


def _repack_body(
    tableT_hbm, tail_hbm, out_hbm,
    ibuf0, ibuf1, obuf0, obuf1, tbuf,
    isem0, isem1, osem0, osem1,
):
    c = lax.axis_index("c")
    s = lax.axis_index("s")
    wid = s * NC + c
    n_w = (NFULL - wid + NW - 1) // NW
    ivec = lax.iota(jnp.int32, 16) * EMBED_DIM
    ibufs, obufs = (ibuf0, ibuf1), (obuf0, obuf1)
    isems, osems = (isem0, isem1), (osem0, osem1)

    def in_cp(j, b):
        ch = wid + j * NW
        return pltpu.make_async_copy(
            tableT_hbm.at[:, pl.ds(ch * CCOL, CCOL)], ibufs[b], isems[b]
        )

    def out_cp(j, b):
        ch = wid + j * NW
        return pltpu.make_async_copy(
            obufs[b],
            out_hbm.at[pl.ds(ch * CHUNK_FLOATS, CHUNK_FLOATS)],
            osems[b],
        )

    def transpose_cols(ib, ob, nk8):
        for cc in range(16):
            idxc = ivec + cc
            def kbody(k8, _):
                for u in range(8):
                    k = k8 * 8 + u
                    xv = ib[cc, pl.ds(k * 16, 16)]
                    idx = idxc + k * 256
                    plsc.store_scatter(ob, [idx], xv)
                return 0

            lax.fori_loop(0, nk8, kbody, 0)

    in_cp(0, 0).start()

    def chunk_pair(j2, _):
        for b in (0, 1):
            j = j2 * 2 + b

            @pl.when(j < n_w)
            def _do():
                in_cp(j, b).wait()

                @pl.when(j + 1 < n_w)
                def _pre():
                    in_cp(j + 1, 1 - b).start()

                @pl.when(j >= 2)
                def _drain():
                    out_cp(j - 2, b).wait()

                transpose_cols(ibufs[b], obufs[b], CCOL // 128)
                out_cp(j, b).start()

        return 0

    lax.fori_loop(0, (81 + 1) // 2, chunk_pair, 0)

    for b in (0, 1):
        for d in (2, 1):
            @pl.when((n_w >= d) & ((n_w - d) % 2 == b))
            def _fin():
                out_cp(n_w - d, b).wait()

    @pl.when(wid == NW - 1)
    def _tail():
        pltpu.async_copy(tail_hbm, tbuf, isem0).wait()
        pltpu.async_copy(
            tbuf, out_hbm.at[pl.ds(TAIL_OFF * EMBED_DIM, 1024)], isem0
        ).wait()


_repack = functools.partial(
    pl.kernel,
    out_type=jax.ShapeDtypeStruct((OUT_FLOATS,), jnp.float32),
    mesh=plsc.VectorSubcoreMesh(core_axis_name="c", subcore_axis_name="s"),
    scratch_types=[
        pltpu.VMEM((16, CCOL), jnp.float32),
        pltpu.VMEM((16, CCOL), jnp.float32),
        pltpu.VMEM((CHUNK_FLOATS,), jnp.float32),
        pltpu.VMEM((CHUNK_FLOATS,), jnp.float32),
        pltpu.VMEM((1024,), jnp.float32),
        pltpu.SemaphoreType.DMA,
        pltpu.SemaphoreType.DMA,
        pltpu.SemaphoreType.DMA,
        pltpu.SemaphoreType.DMA,
    ],
    compiler_params=pltpu.CompilerParams(needs_layout_passes=False),
)(_repack_body)
CI = 128
ROWS_PER_W = BF // (NW * CI)
KIN = 13
NOUT = ROWS_PER_W // KIN
assert NOUT * KIN * CI * NW == BF


def _gather_body(idx_hbm, table_hbm, out_hbm, idx_v, rows_v, sem):
    c = lax.axis_index("c")
    s = lax.axis_index("s")
    wid = s * NC + c
    row0 = wid * ROWS_PER_W
    pltpu.sync_copy(idx_hbm.at[pl.ds(row0, ROWS_PER_W)], idx_v)

    def outer(o, carry):
        base_row = o * KIN
        cps = [
            pltpu.async_copy(
                table_hbm.at[idx_v.at[base_row + j]],
                rows_v.at[pl.ds(j * CI, CI)],
                sem,
            )
            for j in range(KIN)
        ]
        for cp in cps:
            cp.wait()
        out_row0 = row0 * CI + o * (KIN * CI)
        pltpu.sync_copy(rows_v, out_hbm.at[pl.ds(out_row0, KIN * CI)])
        return carry

    lax.fori_loop(0, NOUT, outer, 0)


_gather = functools.partial(
    pl.kernel,
    out_type=jax.ShapeDtypeStruct((BF, EMBED_DIM), jnp.float32),
    mesh=plsc.VectorSubcoreMesh(core_axis_name="c", subcore_axis_name="s"),
    scratch_types=[
        pltpu.VMEM((ROWS_PER_W, CI), jnp.int32),
        pltpu.VMEM((KIN * CI, EMBED_DIM), jnp.float32),
        pltpu.SemaphoreType.DMA,
    ],
    compiler_params=pltpu.CompilerParams(use_tc_tiling_on_sc=False),
)(_gather_body)


BM = 1024


def _mlp_body(flat_ref, w1_ref, b1_ref, w2_ref, b2_ref, out_ref):
    h = jnp.dot(flat_ref[...], w1_ref[...], preferred_element_type=jnp.float32)
    h = jnp.maximum(h + b1_ref[...][None, :], 0.0)
    logit = jnp.sum(h * w2_ref[...][None, :], axis=1) + b2_ref[0]
    out_ref[...] = jax.nn.sigmoid(logit)


def _mlp(flat, w1, b1, w2, b2):
    grid = (BATCH // BM,)
    return pl.pallas_call(
        _mlp_body,
        grid=grid,
        in_specs=[
            pl.BlockSpec((BM, EMBED_OUT), lambda i: (i, 0)),
            pl.BlockSpec((EMBED_OUT, MLP_HIDDEN), lambda i: (0, 0)),
            pl.BlockSpec((MLP_HIDDEN,), lambda i: (0,)),
            pl.BlockSpec((MLP_HIDDEN,), lambda i: (0,)),
            pl.BlockSpec((1,), lambda i: (0,)),
        ],
        out_specs=pl.BlockSpec((BM,), lambda i: (i,)),
        out_shape=jax.ShapeDtypeStruct((BATCH,), jnp.float32),
    )(flat, w1, b1, w2, b2)


def kernel(x, table, W1, b1, W2, b2):
    offsets = jnp.arange(NUM_FIELDS, dtype=jnp.int32) * VOCAB_PER_FIELD
    idx = (x + offsets[None, :]).reshape(BF // CI, CI)
    tail128 = table[TAIL_OFF:].reshape(1024)
    t128 = _repack(table.T, tail128)
    table_rm = t128.reshape(ROWS_PAD, EMBED_DIM)
    gathered = _gather(idx, table_rm)
    flat = gathered.reshape(BATCH, EMBED_OUT)
    return _mlp(flat, W1, b1, W2.reshape(MLP_HIDDEN), b2)


# speedup vs baseline: 1.0170x; 1.0170x over previous
import functools

import jax
import jax.numpy as jnp
import numpy as np
from jax import lax
from jax.experimental import pallas as pl
from jax.experimental.pallas import tpu as pltpu
from jax.experimental.pallas import tpu_sc as plsc

NUM_FIELDS = 26
VOCAB_PER_FIELD = 100000
EMBED_DIM = 16
BATCH = 16384
MLP_HIDDEN = 128
EMBED_OUT = NUM_FIELDS * EMBED_DIM
TOTAL_ROWS = NUM_FIELDS * VOCAB_PER_FIELD
BF = BATCH * NUM_FIELDS

NC = 2
NS = 16
NW = NC * NS

CCOL = 1024
NFULL = 2539
TAIL_OFF = NFULL * CCOL
TAILC = 64
ROWS_PAD = TOTAL_ROWS + TAILC
CHUNK_FLOATS = CCOL * EMBED_DIM


OUT_FLOATS = ROWS_PAD * EMBED_DIM


def _repack_body(
    tableT_hbm, tail_hbm, out_hbm,
    ibuf0, ibuf1, obuf0, obuf1, tbuf,
    isem0, isem1, osem0, osem1,
):
    c = lax.axis_index("c")
    s = lax.axis_index("s")
    wid = s * NC + c
    n_w = (NFULL - wid + NW - 1) // NW
    ivec = lax.iota(jnp.int32, 16) * EMBED_DIM
    ibufs, obufs = (ibuf0, ibuf1), (obuf0, obuf1)
    isems, osems = (isem0, isem1), (osem0, osem1)

    def in_cp(j, b):
        ch = wid + j * NW
        return pltpu.make_async_copy(
            tableT_hbm.at[:, pl.ds(ch * CCOL, CCOL)], ibufs[b], isems[b]
        )

    def out_cp(j, b):
        ch = wid + j * NW
        return pltpu.make_async_copy(
            obufs[b],
            out_hbm.at[pl.ds(ch * CHUNK_FLOATS, CHUNK_FLOATS)],
            osems[b],
        )

    def transpose_cols(ib, ob, nk4):
        for cc in range(16):
            idxc = ivec + cc
            def kbody(k4, _):
                for u in range(4):
                    k = k4 * 4 + u
                    xv = ib[cc, pl.ds(k * 16, 16)]
                    idx = idxc + k * 256
                    plsc.store_scatter(ob, [idx], xv)
                return 0

            lax.fori_loop(0, nk4, kbody, 0)

    in_cp(0, 0).start()

    def chunk_pair(j2, _):
        for b in (0, 1):
            j = j2 * 2 + b

            @pl.when(j < n_w)
            def _do():
                in_cp(j, b).wait()

                @pl.when(j + 1 < n_w)
                def _pre():
                    in_cp(j + 1, 1 - b).start()

                @pl.when(j >= 2)
                def _drain():
                    out_cp(j - 2, b).wait()

                transpose_cols(ibufs[b], obufs[b], CCOL // 64)
                out_cp(j, b).start()

        return 0

    lax.fori_loop(0, (81 + 1) // 2, chunk_pair, 0)

    for b in (0, 1):
        for d in (2, 1):
            @pl.when((n_w >= d) & ((n_w - d) % 2 == b))
            def _fin():
                out_cp(n_w - d, b).wait()

    @pl.when(wid == NW - 1)
    def _tail():
        pltpu.async_copy(tail_hbm, tbuf, isem0).wait()
        pltpu.async_copy(
            tbuf, out_hbm.at[pl.ds(TAIL_OFF * EMBED_DIM, 1024)], isem0
        ).wait()


_repack = functools.partial(
    pl.kernel,
    out_type=jax.ShapeDtypeStruct((OUT_FLOATS,), jnp.float32),
    mesh=plsc.VectorSubcoreMesh(core_axis_name="c", subcore_axis_name="s"),
    scratch_types=[
        pltpu.VMEM((16, CCOL), jnp.float32),
        pltpu.VMEM((16, CCOL), jnp.float32),
        pltpu.VMEM((CHUNK_FLOATS,), jnp.float32),
        pltpu.VMEM((CHUNK_FLOATS,), jnp.float32),
        pltpu.VMEM((1024,), jnp.float32),
        pltpu.SemaphoreType.DMA,
        pltpu.SemaphoreType.DMA,
        pltpu.SemaphoreType.DMA,
        pltpu.SemaphoreType.DMA,
    ],
    compiler_params=pltpu.CompilerParams(needs_layout_passes=False),
)(_repack_body)
CI = 128
ROWS_PER_W = BF // (NW * CI)
KIN = 13
NOUT = ROWS_PER_W // KIN
assert NOUT * KIN * CI * NW == BF


def _gather_body(idx_hbm, table_hbm, out_hbm, idx_v, rows_v, sem):
    c = lax.axis_index("c")
    s = lax.axis_index("s")
    wid = s * NC + c
    row0 = wid * ROWS_PER_W
    pltpu.sync_copy(idx_hbm.at[pl.ds(row0, ROWS_PER_W)], idx_v)

    def outer(o, carry):
        base_row = o * KIN
        cps = [
            pltpu.async_copy(
                table_hbm.at[idx_v.at[base_row + j]],
                rows_v.at[pl.ds(j * CI, CI)],
                sem,
            )
            for j in range(KIN)
        ]
        for cp in cps:
            cp.wait()
        out_row0 = row0 * CI + o * (KIN * CI)
        pltpu.sync_copy(rows_v, out_hbm.at[pl.ds(out_row0, KIN * CI)])
        return carry

    lax.fori_loop(0, NOUT, outer, 0)


_gather = functools.partial(
    pl.kernel,
    out_type=jax.ShapeDtypeStruct((BF, EMBED_DIM), jnp.float32),
    mesh=plsc.VectorSubcoreMesh(core_axis_name="c", subcore_axis_name="s"),
    scratch_types=[
        pltpu.VMEM((ROWS_PER_W, CI), jnp.int32),
        pltpu.VMEM((KIN * CI, EMBED_DIM), jnp.float32),
        pltpu.SemaphoreType.DMA,
    ],
    compiler_params=pltpu.CompilerParams(use_tc_tiling_on_sc=False),
)(_gather_body)


BM = 1024


def _mlp_body(flat_ref, w1_ref, b1_ref, w2_ref, b2_ref, out_ref):
    h = jnp.dot(flat_ref[...], w1_ref[...], preferred_element_type=jnp.float32)
    h = jnp.maximum(h + b1_ref[...][None, :], 0.0)
    logit = jnp.sum(h * w2_ref[...][None, :], axis=1) + b2_ref[0]
    out_ref[...] = jax.nn.sigmoid(logit)


def _mlp(flat, w1, b1, w2, b2):
    grid = (BATCH // BM,)
    return pl.pallas_call(
        _mlp_body,
        grid=grid,
        in_specs=[
            pl.BlockSpec((BM, EMBED_OUT), lambda i: (i, 0)),
            pl.BlockSpec((EMBED_OUT, MLP_HIDDEN), lambda i: (0, 0)),
            pl.BlockSpec((MLP_HIDDEN,), lambda i: (0,)),
            pl.BlockSpec((MLP_HIDDEN,), lambda i: (0,)),
            pl.BlockSpec((1,), lambda i: (0,)),
        ],
        out_specs=pl.BlockSpec((BM,), lambda i: (i,)),
        out_shape=jax.ShapeDtypeStruct((BATCH,), jnp.float32),
    )(flat, w1, b1, w2, b2)


def kernel(x, table, W1, b1, W2, b2):
    offsets = jnp.arange(NUM_FIELDS, dtype=jnp.int32) * VOCAB_PER_FIELD
    idx = (x + offsets[None, :]).reshape(BF // CI, CI)
    tail128 = table[TAIL_OFF:].reshape(1024)
    t128 = _repack(table.T, tail128)
    table_rm = t128.reshape(ROWS_PAD, EMBED_DIM)
    gathered = _gather(idx, table_rm)
    flat = gathered.reshape(BATCH, EMBED_OUT)
    return _mlp(flat, W1, b1, W2.reshape(MLP_HIDDEN), b2)
